# Initial kernel scaffold; baseline (speedup 1.0000x reference)
import jax
import jax.numpy as jnp
from jax.experimental import pallas as pl


def kernel(text_indices, numeric_values, embedding_table):
    raise NotImplementedError("write your pallas kernel here")


# speedup vs baseline: 155.1868x; 155.1868x over previous
import functools

import jax
import jax.numpy as jnp
from jax import lax
from jax.experimental import pallas as pl
from jax.experimental.pallas import tpu as pltpu
from jax.experimental.pallas import tpu_sc as plsc

_B = 16384
_T = 200
_N = 26
_V = 1_000_000
_NC = 2
_NS = 16
_NW = _NC * _NS
_RPW = _B // _NW
_CH = 4
_R = _RPW // _CH
_TCHUNK = 62528

_BT = _B * _T
_EPW = _BT // _NW
_E = _EPW // _CH


def _gather_body(idx_hbm, tab_hbm, out_hbm, tab_sh, idx_v, vals_v, sem):
    cid = lax.axis_index("c")
    sid = lax.axis_index("s")
    wid = sid * _NC + cid

    start = jnp.minimum(sid * _TCHUNK, _V - _TCHUNK)
    done = 0
    while done < _TCHUNK:
        step = min(_E, _TCHUNK - done)
        s = start + done
        pltpu.sync_copy(tab_hbm.at[pl.ds(s, step)], vals_v.at[pl.ds(0, step)])
        pltpu.sync_copy(vals_v.at[pl.ds(0, step)], tab_sh.at[pl.ds(s, step)])
        done += step
    plsc.subcore_barrier()

    base = wid * _EPW
    for c in range(_CH):
        off = base + c * _E
        pltpu.sync_copy(idx_hbm.at[pl.ds(off, _E)], idx_v)
        pltpu.async_copy(tab_sh.at[idx_v], vals_v, sem).wait()
        pltpu.sync_copy(vals_v, out_hbm.at[pl.ds(off, _E)])


@functools.cache
def _gather_sc():
    mesh = plsc.VectorSubcoreMesh(
        core_axis_name="c", subcore_axis_name="s",
        num_cores=_NC, num_subcores=_NS,
    )
    return pl.kernel(
        _gather_body,
        out_type=jax.ShapeDtypeStruct((_BT,), jnp.float32),
        mesh=mesh,
        scratch_types=[
            pltpu.VMEM_SHARED((_V,), jnp.float32),
            pltpu.VMEM((_E,), jnp.int32),
            pltpu.VMEM((_E,), jnp.float32),
            pltpu.SemaphoreType.DMA,
        ],
    )


_SB = 512
_SG = _B // _SB


def _stats_body(num_ref, stat_ref, acc_ref):
    i = pl.program_id(0)

    @pl.when(i == 0)
    def _init():
        acc_ref[0] = 0.0
        acc_ref[1] = 0.0

    col0 = num_ref[:, 0:1]
    acc_ref[0] += jnp.sum(col0)
    acc_ref[1] += jnp.sum(col0 * col0)

    @pl.when(i == _SG - 1)
    def _fin():
        s = acc_ref[0]
        ss = acc_ref[1]
        mean = s / _B
        var = (ss - s * s / _B) / (_B - 1)
        inv = lax.rsqrt(var)
        r = lax.broadcasted_iota(jnp.int32, (8, 128), 0)
        c = lax.broadcasted_iota(jnp.int32, (8, 128), 1)
        first = (r == 0) & (c == 0)
        second = (r == 0) & (c == 1)
        stat_ref[...] = jnp.where(first, mean, jnp.where(second, inv, 0.0))


_stats_call = pl.pallas_call(
    _stats_body,
    grid=(_SG,),
    in_specs=[pl.BlockSpec((_SB, _N), lambda i: (i, 0))],
    out_specs=pl.BlockSpec((8, 128), lambda i: (0, 0)),
    out_shape=jax.ShapeDtypeStruct((8, 128), jnp.float32),
    scratch_shapes=[pltpu.SMEM((2,), jnp.float32)],
)


def _finish_body(stat_ref, num_ref, outn_ref, mask_ref):
    mean = stat_ref[0, 0]
    inv = stat_ref[0, 1]
    x = num_ref[...]
    col = lax.broadcasted_iota(jnp.int32, x.shape, 1)
    y = jnp.where(col == 0, (x - mean) * inv, x)
    nan = jnp.isnan(y)
    outn_ref[...] = jnp.where(nan, 0.0, y)
    mask_ref[:, : _T] = jnp.ones((x.shape[0], _T), jnp.bool_)
    mask_ref[:, _T:] = ~nan


_finish_call = pl.pallas_call(
    _finish_body,
    grid=(_SG,),
    in_specs=[
        pl.BlockSpec((8, 128), lambda i: (0, 0)),
        pl.BlockSpec((_SB, _N), lambda i: (i, 0)),
    ],
    out_specs=[
        pl.BlockSpec((_SB, _N), lambda i: (i, 0)),
        pl.BlockSpec((_SB, _T + _N), lambda i: (i, 0)),
    ],
    out_shape=[
        jax.ShapeDtypeStruct((_B, _N), jnp.float32),
        jax.ShapeDtypeStruct((_B, _T + _N), jnp.bool_),
    ],
)


def kernel(text_indices, numeric_values, embedding_table):
    tab = embedding_table.reshape(_V)
    idx = text_indices.astype(jnp.int32).reshape(_BT)
    out_text = _gather_sc()(idx, tab).reshape(_B, _T)
    stats = _stats_call(numeric_values)
    out_num, mask = _finish_call(stats, numeric_values)
    out = jnp.concatenate([out_text, out_num], axis=1)[:, :, None]
    return out, mask[:, :, None]
